# baseline (speedup 1.0000x reference)
import jax
import jax.numpy as jnp
from jax.experimental import pallas as pl
from jax.experimental.pallas import tpu as pltpu

CODE_DIM = 32
N_CENT = 512
D = 1024
TILE_W = 512
TILE_M = 256
AUG = 64
N_QSTEPS = 2 * D // TILE_W


def _fused_kernel(v2_ref, m_ref, b_ref, ca_ref, x_ref, o_ref, w_ref):
    i = pl.program_id(0)

    @pl.when(i < N_QSTEPS)
    def _quantize():
        base = i * TILE_W
        for j in range(CODE_DIM):
            vj = v2_ref[:, CODE_DIM * j:CODE_DIM * (j + 1)].astype(jnp.bfloat16)
            logits = jax.lax.dot_general(
                vj, m_ref[...], (((1,), (0,)), ((), ())),
                preferred_element_type=jnp.float32) + b_ref[...]
            e = jnp.exp(logits).astype(jnp.bfloat16)
            qs = jax.lax.dot_general(
                e, ca_ref[...], (((1,), (0,)), ((), ())),
                preferred_element_type=jnp.float32)
            w_ref[pl.ds(base, TILE_W), CODE_DIM * j:CODE_DIM * (j + 1)] = (
                qs[:, :CODE_DIM] * (1.0 / qs[:, CODE_DIM:])
            ).astype(jnp.bfloat16)

    @pl.when(i >= N_QSTEPS)
    def _forward():
        h = jnp.maximum(
            jnp.dot(x_ref[...].astype(jnp.bfloat16), w_ref[:D, :],
                    preferred_element_type=jnp.float32),
            0.0)
        o_ref[...] = jnp.dot(h.astype(jnp.bfloat16), w_ref[D:, :],
                             preferred_element_type=jnp.float32)


def kernel(x, subspace_params, centroids):
    v2 = subspace_params.reshape(2 * D, D)
    m = (2.0 * centroids.T).astype(jnp.bfloat16)
    b = -jnp.sum(centroids * centroids, axis=-1)[None, :]
    ca = jnp.concatenate(
        [centroids, jnp.ones((N_CENT, AUG - CODE_DIM), jnp.float32)],
        axis=1).astype(jnp.bfloat16)
    n_msteps = x.shape[0] // TILE_M

    out = pl.pallas_call(
        _fused_kernel,
        grid=(N_QSTEPS + n_msteps,),
        in_specs=[
            pl.BlockSpec((TILE_W, D), lambda i: (jnp.minimum(i, N_QSTEPS - 1), 0)),
            pl.BlockSpec((CODE_DIM, N_CENT), lambda i: (0, 0)),
            pl.BlockSpec((1, N_CENT), lambda i: (0, 0)),
            pl.BlockSpec((N_CENT, AUG), lambda i: (0, 0)),
            pl.BlockSpec((TILE_M, D),
                         lambda i: (jnp.maximum(i - N_QSTEPS, 0), 0)),
        ],
        out_specs=pl.BlockSpec((TILE_M, D),
                               lambda i: (jnp.maximum(i - N_QSTEPS, 0), 0)),
        out_shape=jax.ShapeDtypeStruct((x.shape[0], D), jnp.float32),
        scratch_shapes=[pltpu.VMEM((2 * D, D), jnp.bfloat16)],
    )(v2, m, b, ca, x)
    return out


# speedup vs baseline: 1.0262x; 1.0262x over previous
import jax
import jax.numpy as jnp
from jax.experimental import pallas as pl
from jax.experimental.pallas import tpu as pltpu

CODE_DIM = 32
N_CENT = 512
D = 1024
TILE_W = 512
TILE_M = 512
AUG = 64
N_QSTEPS = 2 * D // TILE_W


def _fused_kernel(v2_ref, m_ref, b_ref, ca_ref, x_ref, o_ref, w_ref):
    i = pl.program_id(0)

    @pl.when(i < N_QSTEPS)
    def _quantize():
        base = i * TILE_W
        for j in range(CODE_DIM):
            vj = v2_ref[:, CODE_DIM * j:CODE_DIM * (j + 1)].astype(jnp.bfloat16)
            logits = jax.lax.dot_general(
                vj, m_ref[...], (((1,), (0,)), ((), ())),
                preferred_element_type=jnp.float32) + b_ref[...]
            e = jnp.exp(logits).astype(jnp.bfloat16)
            qs = jax.lax.dot_general(
                e, ca_ref[...], (((1,), (0,)), ((), ())),
                preferred_element_type=jnp.float32)
            w_ref[pl.ds(base, TILE_W), CODE_DIM * j:CODE_DIM * (j + 1)] = (
                qs[:, :CODE_DIM] * (1.0 / qs[:, CODE_DIM:])
            ).astype(jnp.bfloat16)

    @pl.when(i >= N_QSTEPS)
    def _forward():
        h = jnp.maximum(
            jnp.dot(x_ref[...].astype(jnp.bfloat16), w_ref[:D, :],
                    preferred_element_type=jnp.float32),
            0.0)
        o_ref[...] = jnp.dot(h.astype(jnp.bfloat16), w_ref[D:, :],
                             preferred_element_type=jnp.float32)


def kernel(x, subspace_params, centroids):
    v2 = subspace_params.reshape(2 * D, D)
    m = (2.0 * centroids.T).astype(jnp.bfloat16)
    b = -jnp.sum(centroids * centroids, axis=-1)[None, :]
    ca = jnp.concatenate(
        [centroids, jnp.ones((N_CENT, AUG - CODE_DIM), jnp.float32)],
        axis=1).astype(jnp.bfloat16)
    n_msteps = x.shape[0] // TILE_M

    out = pl.pallas_call(
        _fused_kernel,
        grid=(N_QSTEPS + n_msteps,),
        in_specs=[
            pl.BlockSpec((TILE_W, D), lambda i: (jnp.minimum(i, N_QSTEPS - 1), 0)),
            pl.BlockSpec((CODE_DIM, N_CENT), lambda i: (0, 0)),
            pl.BlockSpec((1, N_CENT), lambda i: (0, 0)),
            pl.BlockSpec((N_CENT, AUG), lambda i: (0, 0)),
            pl.BlockSpec((TILE_M, D),
                         lambda i: (jnp.maximum(i - N_QSTEPS, 0), 0)),
        ],
        out_specs=pl.BlockSpec((TILE_M, D),
                               lambda i: (jnp.maximum(i - N_QSTEPS, 0), 0)),
        out_shape=jax.ShapeDtypeStruct((x.shape[0], D), jnp.float32),
        scratch_shapes=[pltpu.VMEM((2 * D, D), jnp.bfloat16)],
    )(v2, m, b, ca, x)
    return out
